# baseline (speedup 1.0000x reference)
import functools

import jax
import jax.numpy as jnp
from jax import lax
from jax.experimental import pallas as pl
from jax.experimental.pallas import tpu as pltpu
from jax.experimental.pallas import tpu_sc as plsc

N_CORES = 2
N_SUBCORES = 16
NW = N_CORES * N_SUBCORES
LANES = 16
BATCH = 16384
EMB_D = 32
RPW = BATCH // NW
CHUNK = 128
NCH = RPW // CHUNK
MARGIN = 0.5
ALPHA = 0.1
GROUPS = RPW // LANES
NB = BATCH // CHUNK


def _sc_loss_body(user_emb, item_emb, un4, pn4, nn4,
                  uid_hbm, pid_hbm, nid_hbm, out_hbm,
                  uidx_v, pidx_v, nidx_v,
                  u_v, p_v, n_v, un_v, pn_v, nn_v, out_v, sem):
    w = lax.axis_index("s") * N_CORES + lax.axis_index("c")

    pltpu.sync_copy(uid_hbm.at[w], uidx_v)
    pltpu.sync_copy(pid_hbm.at[w], pidx_v)
    pltpu.sync_copy(nid_hbm.at[w], nidx_v)

    cblocks = RPW // CHUNK
    copies = []
    for j in range(NCH):
        sl = pl.ds(j * CHUNK, CHUNK)
        copies.append(pltpu.async_copy(user_emb.at[uidx_v.at[j]], u_v.at[sl], sem))
        copies.append(pltpu.async_copy(item_emb.at[pidx_v.at[j]], p_v.at[sl], sem))
        copies.append(pltpu.async_copy(item_emb.at[nidx_v.at[j]], n_v.at[sl], sem))
    nsl = pl.ds(w * cblocks, cblocks)
    copies.append(pltpu.async_copy(un4.at[:, nsl], un_v, sem))
    copies.append(pltpu.async_copy(pn4.at[:, nsl], pn_v, sem))
    copies.append(pltpu.async_copy(nn4.at[:, nsl], nn_v, sem))
    for c in copies:
        c.wait()

    iota = lax.iota(jnp.int32, LANES)
    zero = jnp.zeros((LANES,), jnp.float32)
    gpb = CHUNK // LANES

    def group(g, carry):
        hingeacc, regacc, pdacc = carry
        row = g * LANES + iota
        cb = g // gpb
        c0 = (g % gpb) * LANES
        hacc = zero
        for d in range(EMB_D):
            dr, dc = d // 8, d % 8
            col = jnp.full((LANES,), d, jnp.int32)
            u = plsc.load_gather(u_v, [row, col])
            p = plsc.load_gather(p_v, [row, col])
            n = plsc.load_gather(n_v, [row, col])
            un = un_v[dr, cb, dc, pl.ds(c0, LANES)]
            pn = pn_v[dr, cb, dc, pl.ds(c0, LANES)]
            nn = nn_v[dr, cb, dc, pl.ds(c0, LANES)]
            dp = u + pn * un - p
            dn = u + nn * un - n
            dp2 = dp * dp
            hacc = hacc + dp2 - dn * dn
            pdacc = pdacc + dp2
            du = u - un
            regacc = regacc + du * du
            dpp = p - pn
            regacc = regacc + dpp * dpp
            dnn = n - nn
            regacc = regacc + dnn * dnn
        hingeacc = hingeacc + jnp.maximum(hacc + MARGIN, 0.0)
        return hingeacc, regacc, pdacc

    hingeacc, regacc, pdacc = lax.fori_loop(0, GROUPS, group, (zero, zero, zero))
    out_v[...] = hingeacc + ALPHA * (regacc + pdacc)
    pltpu.sync_copy(out_v, out_hbm.at[w])


@functools.partial(
    pl.kernel,
    out_type=jax.ShapeDtypeStruct((NW, LANES), jnp.float32),
    mesh=plsc.VectorSubcoreMesh(core_axis_name="c", subcore_axis_name="s"),
    scratch_types=[
        pltpu.VMEM((NCH, CHUNK), jnp.int32),
        pltpu.VMEM((NCH, CHUNK), jnp.int32),
        pltpu.VMEM((NCH, CHUNK), jnp.int32),
        pltpu.VMEM((RPW, EMB_D), jnp.float32),
        pltpu.VMEM((RPW, EMB_D), jnp.float32),
        pltpu.VMEM((RPW, EMB_D), jnp.float32),
        pltpu.VMEM((4, RPW // CHUNK, 8, CHUNK), jnp.float32),
        pltpu.VMEM((4, RPW // CHUNK, 8, CHUNK), jnp.float32),
        pltpu.VMEM((4, RPW // CHUNK, 8, CHUNK), jnp.float32),
        pltpu.VMEM((LANES,), jnp.float32),
        pltpu.SemaphoreType.DMA,
    ],
    compiler_params=pltpu.CompilerParams(
        needs_layout_passes=False, use_tc_tiling_on_sc=False),
)
def _sc_loss(user_emb, item_emb, un4, pn4, nn4,
             uid_hbm, pid_hbm, nid_hbm, out_hbm, *scratch):
    _sc_loss_body(user_emb, item_emb, un4, pn4, nn4,
                  uid_hbm, pid_hbm, nid_hbm, out_hbm, *scratch)


def kernel(user_embeddings, item_embeddings, user_neighbors, pos_neighbors,
           neg_neighbors, user_ids, pos_ids, neg_ids):
    un4 = user_neighbors.T.reshape(4, 8, NB, CHUNK).transpose(0, 2, 1, 3)
    pn4 = pos_neighbors.T.reshape(4, 8, NB, CHUNK).transpose(0, 2, 1, 3)
    nn4 = neg_neighbors.T.reshape(4, 8, NB, CHUNK).transpose(0, 2, 1, 3)
    uid3 = user_ids.astype(jnp.int32).reshape(NW, NCH, CHUNK)
    pid3 = pos_ids.astype(jnp.int32).reshape(NW, NCH, CHUNK)
    nid3 = neg_ids.astype(jnp.int32).reshape(NW, NCH, CHUNK)
    partials = _sc_loss(user_embeddings, item_embeddings, un4, pn4, nn4,
                        uid3, pid3, nid3)
    return jnp.sum(partials)
